# baseline (speedup 1.0000x reference)
import functools

import jax
import jax.numpy as jnp
from jax import lax
from jax.experimental import pallas as pl
from jax.experimental.pallas import tpu as pltpu
from jax.experimental.pallas import tpu_sc as plsc

N = 10000
E = 320000
F = 128
R = 16

ALPHA = 0.01
MU = 2.5

NC = 2
NS = 16
NW = NC * NS
EH = E // 2
EPW = EH // NW
CHUNK = 40
NCHUNK = EPW // CHUNK
NPAD = 10240
ROWS_PER_S = NPAD // NS


def _smu(x):
    return ((1 + ALPHA) * x
            + (1 - ALPHA) * x * lax.erf(MU * (1 - ALPHA) * x)) / 2


def _h_body(x_ref, w_ref, o_ref):
    o_ref[...] = jnp.dot(x_ref[...], w_ref[...],
                         preferred_element_type=jnp.float32)


def _compute_h(x, w):
    blk = 2000
    return pl.pallas_call(
        _h_body,
        grid=(N // blk,),
        in_specs=[
            pl.BlockSpec((blk, F), lambda i: (i, 0)),
            pl.BlockSpec((F, F), lambda i: (0, 0)),
        ],
        out_specs=pl.BlockSpec((blk, F), lambda i: (i, 0)),
        out_shape=jax.ShapeDtypeStruct((N, F), jnp.float32),
    )(x, w)


def _wij_body(f_ref, rc_ref, w1_ref, b1_ref, w2_ref, b2_ref, o_ref):
    t = jnp.dot(f_ref[...], w1_ref[...], preferred_element_type=jnp.float32)
    t = _smu(t + b1_ref[...])
    w = jnp.dot(t, w2_ref[...], preferred_element_type=jnp.float32)
    o_ref[...] = (w + b2_ref[...]) * rc_ref[...]


def _compute_wij(f_ij, rcut, w1, b1, w2, b2, e0):
    blk = 4000
    b0 = e0 // blk
    return pl.pallas_call(
        _wij_body,
        grid=(EH // blk,),
        in_specs=[
            pl.BlockSpec((blk, R), lambda i: (i + b0, 0)),
            pl.BlockSpec((blk, 1), lambda i: (i + b0, 0)),
            pl.BlockSpec((R, F), lambda i: (0, 0)),
            pl.BlockSpec((1, F), lambda i: (0, 0)),
            pl.BlockSpec((F, F), lambda i: (0, 0)),
            pl.BlockSpec((1, F), lambda i: (0, 0)),
        ],
        out_specs=pl.BlockSpec((blk, F), lambda i: (i, 0)),
        out_shape=jax.ShapeDtypeStruct((EH, F), jnp.float32),
    )(f_ij, rcut, w1, b1, w2, b2)


def _mul_rows(xj, wij):
    def row_body(r, c2):
        for k in range(F // 16):
            sl = pl.ds(k * 16, 16)
            xj[r, sl] = xj[r, sl] * wij[r, sl]
        return c2
    lax.fori_loop(0, CHUNK, row_body, 0)


def _sc_body(h_hbm, wij_hbm, idx_hbm, init_hbm, out_hbm,
             idx_v, xj_a, wij_a, xj_b, wij_b, agg_sh,
             g_a, w_a, s_a, g_b, w_b, s_b):
    cid = lax.axis_index("c")
    sid = lax.axis_index("s")
    wid = sid * NC + cid
    base_e = wid * EPW

    rows = pl.ds(sid * ROWS_PER_S, ROWS_PER_S)
    pltpu.sync_copy(init_hbm.at[cid, rows], agg_sh.at[rows])
    plsc.subcore_barrier()

    def pair_body(i, carry):
        ta = 2 * i
        tb = 2 * i + 1
        offa = pl.multiple_of(base_e + ta * CHUNK, 8)
        offb = pl.multiple_of(base_e + tb * CHUNK, 8)
        pltpu.sync_copy(idx_hbm.at[wid, pl.ds(ta, 2)], idx_v)
        cga = pltpu.async_copy(h_hbm.at[idx_v.at[0, 1]], xj_a, g_a)
        cwa = pltpu.async_copy(wij_hbm.at[pl.ds(offa, CHUNK)], wij_a, w_a)
        cgb = pltpu.async_copy(h_hbm.at[idx_v.at[1, 1]], xj_b, g_b)
        cwb = pltpu.async_copy(wij_hbm.at[pl.ds(offb, CHUNK)], wij_b, w_b)
        cga.wait()
        cwa.wait()
        _mul_rows(xj_a, wij_a)
        csa = pltpu.async_copy(xj_a, agg_sh.at[idx_v.at[0, 0]], s_a, add=True)
        cgb.wait()
        cwb.wait()
        _mul_rows(xj_b, wij_b)
        csb = pltpu.async_copy(xj_b, agg_sh.at[idx_v.at[1, 0]], s_b, add=True)
        csa.wait()
        csb.wait()
        return carry

    lax.fori_loop(0, NCHUNK // 2, pair_body, 0)

    tt = NCHUNK - 1
    offt = pl.multiple_of(base_e + tt * CHUNK, 8)
    pltpu.sync_copy(idx_hbm.at[wid, pl.ds(tt - 1, 2)], idx_v)
    cgt = pltpu.async_copy(h_hbm.at[idx_v.at[1, 1]], xj_a, g_a)
    cwt = pltpu.async_copy(wij_hbm.at[pl.ds(offt, CHUNK)], wij_a, w_a)
    cgt.wait()
    cwt.wait()
    _mul_rows(xj_a, wij_a)
    pltpu.async_copy(xj_a, agg_sh.at[idx_v.at[1, 0]], s_a, add=True).wait()

    plsc.subcore_barrier()
    pltpu.sync_copy(agg_sh.at[rows], out_hbm.at[cid, rows])


def _sc_aggregate(h, wij, idx_i, idx_j, init):
    idx_pair = jnp.stack([idx_i.reshape(NW, NCHUNK, CHUNK),
                          idx_j.reshape(NW, NCHUNK, CHUNK)], axis=2)
    mesh = plsc.VectorSubcoreMesh(core_axis_name="c", subcore_axis_name="s")
    k = functools.partial(
        pl.kernel,
        mesh=mesh,
        out_type=jax.ShapeDtypeStruct((NC, NPAD, F), jnp.float32),
        scratch_types=[
            pltpu.VMEM((2, 2, CHUNK), jnp.int32),
            pltpu.VMEM((CHUNK, F), jnp.float32),
            pltpu.VMEM((CHUNK, F), jnp.float32),
            pltpu.VMEM((CHUNK, F), jnp.float32),
            pltpu.VMEM((CHUNK, F), jnp.float32),
            pltpu.VMEM_SHARED((NPAD, F), jnp.float32),
            pltpu.SemaphoreType.DMA,
            pltpu.SemaphoreType.DMA,
            pltpu.SemaphoreType.DMA,
            pltpu.SemaphoreType.DMA,
            pltpu.SemaphoreType.DMA,
            pltpu.SemaphoreType.DMA,
        ],
    )(_sc_body)
    return k(h, wij, idx_pair, init)


def _out_body(p0_ref, p1_ref, w1_ref, b1_ref, w2_ref, b2_ref, o_ref):
    a = p0_ref[...] + p1_ref[...]
    t = _smu(jnp.dot(a, w1_ref[...], preferred_element_type=jnp.float32)
             + b1_ref[...])
    o_ref[...] = jnp.dot(t, w2_ref[...],
                         preferred_element_type=jnp.float32) + b2_ref[...]


def _compute_out(p0, p1, w1, b1, w2, b2):
    blk = 2000
    return pl.pallas_call(
        _out_body,
        grid=(N // blk,),
        in_specs=[
            pl.BlockSpec((blk, F), lambda i: (i, 0)),
            pl.BlockSpec((blk, F), lambda i: (i, 0)),
            pl.BlockSpec((F, F), lambda i: (0, 0)),
            pl.BlockSpec((1, F), lambda i: (0, 0)),
            pl.BlockSpec((F, F), lambda i: (0, 0)),
            pl.BlockSpec((1, F), lambda i: (0, 0)),
        ],
        out_specs=pl.BlockSpec((blk, F), lambda i: (i, 0)),
        out_shape=jax.ShapeDtypeStruct((N, F), jnp.float32),
    )(p0, p1, w1, b1, w2, b2)


def kernel(x, f_ij, rcut_ij, W_in2f, W_f1, b_f1, W_f2, b_f2,
           W_o1, b_o1, W_o2, b_o2, idx_i, idx_j):
    h = _compute_h(x, W_in2f)
    rc2 = rcut_ij.reshape(E, 1)
    b_f1r = b_f1.reshape(1, F)
    b_f2r = b_f2.reshape(1, F)
    ii = idx_i.astype(jnp.int32)
    jj = idx_j.astype(jnp.int32)
    zeros = jnp.zeros((NC, NPAD, F), jnp.float32)

    wij_a = _compute_wij(f_ij, rc2, W_f1, b_f1r, W_f2, b_f2r, 0)
    parts_a = _sc_aggregate(h, wij_a, ii[:EH], jj[:EH], zeros)
    wij_b = _compute_wij(f_ij, rc2, W_f1, b_f1r, W_f2, b_f2r, EH)
    parts_b = _sc_aggregate(h, wij_b, ii[EH:], jj[EH:], parts_a)

    out = _compute_out(parts_b[0], parts_b[1],
                       W_o1, b_o1.reshape(1, F), W_o2, b_o2.reshape(1, F))
    return out


# speedup vs baseline: 1.0992x; 1.0992x over previous
import functools

import jax
import jax.numpy as jnp
from jax import lax
from jax.experimental import pallas as pl
from jax.experimental.pallas import tpu as pltpu
from jax.experimental.pallas import tpu_sc as plsc

N = 10000
E = 320000
F = 128
R = 16

ALPHA = 0.01
MU = 2.5

NC = 2
NS = 16
NW = NC * NS
WBLK = 4000
PR = E // 2
PRW = PR // NW
CH = 40
NCH = PRW // CH
NPAD = 10240
ROWS_PER_S = NPAD // NS


def _smu(x):
    return ((1 + ALPHA) * x
            + (1 - ALPHA) * x * lax.erf(MU * (1 - ALPHA) * x)) / 2


def _h_body(x_ref, w_ref, o_ref):
    o_ref[...] = jnp.dot(x_ref[...], w_ref[...],
                         preferred_element_type=jnp.float32)


def _compute_h(x, w):
    blk = 2000
    return pl.pallas_call(
        _h_body,
        grid=(N // blk,),
        in_specs=[
            pl.BlockSpec((blk, F), lambda i: (i, 0)),
            pl.BlockSpec((F, F), lambda i: (0, 0)),
        ],
        out_specs=pl.BlockSpec((blk, F), lambda i: (i, 0)),
        out_shape=jax.ShapeDtypeStruct((N, F), jnp.float32),
    )(x, w)


def _wij_body(f_ref, rc_ref, w1_ref, b1_ref, w2_ref, b2_ref, o_ref):
    t = jnp.dot(f_ref[...], w1_ref[...], preferred_element_type=jnp.float32)
    t = _smu(t + b1_ref[...])
    w = jnp.dot(t, w2_ref[...], preferred_element_type=jnp.float32)
    w = (w + b2_ref[...]) * rc_ref[...]
    lo = lax.bitcast_convert_type(
        w[:WBLK // 2].astype(jnp.bfloat16), jnp.uint16).astype(jnp.uint32)
    hi = lax.bitcast_convert_type(
        w[WBLK // 2:].astype(jnp.bfloat16), jnp.uint16).astype(jnp.uint32)
    o_ref[...] = lax.bitcast_convert_type(lo | (hi << 16), jnp.int32)


def _compute_wij(f_ij, rcut, w1, b1, w2, b2):
    return pl.pallas_call(
        _wij_body,
        grid=(E // WBLK,),
        in_specs=[
            pl.BlockSpec((WBLK, R), lambda i: (i, 0)),
            pl.BlockSpec((WBLK, 1), lambda i: (i, 0)),
            pl.BlockSpec((R, F), lambda i: (0, 0)),
            pl.BlockSpec((1, F), lambda i: (0, 0)),
            pl.BlockSpec((F, F), lambda i: (0, 0)),
            pl.BlockSpec((1, F), lambda i: (0, 0)),
        ],
        out_specs=pl.BlockSpec((WBLK // 2, F), lambda i: (i, 0)),
        out_shape=jax.ShapeDtypeStruct((PR, F), jnp.int32),
    )(f_ij, rcut, w1, b1, w2, b2)


def _mul_rows(xj, wp):
    def m_body(m, c2):
        for k in range(F // 16):
            sl = pl.ds(16 * k, 16)
            wv = wp[m, sl]
            a = lax.bitcast_convert_type(wv << 16, jnp.float32)
            b = lax.bitcast_convert_type(wv & jnp.int32(-65536), jnp.float32)
            xj[m, sl] = xj[m, sl] * a
            xj[CH + m, sl] = xj[CH + m, sl] * b
        return c2
    lax.fori_loop(0, CH, m_body, 0)


def _chunk_start(h_hbm, wij_hbm, idx_v, p, off, xj, wp, g, w):
    cg1 = pltpu.async_copy(h_hbm.at[idx_v.at[p, 1]], xj.at[pl.ds(0, CH)], g)
    cg2 = pltpu.async_copy(h_hbm.at[idx_v.at[p, 3]], xj.at[pl.ds(CH, CH)], g)
    cw = pltpu.async_copy(wij_hbm.at[pl.ds(off, CH)], wp, w)
    return cg1, cg2, cw


def _chunk_scatter(agg_sh, idx_v, p, xj, s):
    cs1 = pltpu.async_copy(xj.at[pl.ds(0, CH)], agg_sh.at[idx_v.at[p, 0]],
                           s, add=True)
    cs2 = pltpu.async_copy(xj.at[pl.ds(CH, CH)], agg_sh.at[idx_v.at[p, 2]],
                           s, add=True)
    return cs1, cs2


def _sc_body(h_hbm, wij_hbm, idx_hbm, zeros_hbm, out_hbm,
             idx_v, xj_a, wp_a, xj_b, wp_b, agg_sh,
             g_a, w_a, s_a, g_b, w_b, s_b):
    cid = lax.axis_index("c")
    sid = lax.axis_index("s")
    wid = sid * NC + cid
    base_p = wid * PRW

    rows = pl.ds(sid * ROWS_PER_S, ROWS_PER_S)
    pltpu.sync_copy(zeros_hbm.at[rows], agg_sh.at[rows])
    plsc.subcore_barrier()

    def pair_body(i, carry):
        ta = 2 * i
        tb = 2 * i + 1
        offa = pl.multiple_of(base_p + ta * CH, 8)
        offb = pl.multiple_of(base_p + tb * CH, 8)
        pltpu.sync_copy(idx_hbm.at[wid, pl.ds(ta, 2)], idx_v)
        cga1, cga2, cwa = _chunk_start(h_hbm, wij_hbm, idx_v, 0, offa,
                                       xj_a, wp_a, g_a, w_a)
        cgb1, cgb2, cwb = _chunk_start(h_hbm, wij_hbm, idx_v, 1, offb,
                                       xj_b, wp_b, g_b, w_b)
        cga1.wait()
        cga2.wait()
        cwa.wait()
        _mul_rows(xj_a, wp_a)
        csa1, csa2 = _chunk_scatter(agg_sh, idx_v, 0, xj_a, s_a)
        cgb1.wait()
        cgb2.wait()
        cwb.wait()
        _mul_rows(xj_b, wp_b)
        csb1, csb2 = _chunk_scatter(agg_sh, idx_v, 1, xj_b, s_b)
        csa1.wait()
        csa2.wait()
        csb1.wait()
        csb2.wait()
        return carry

    lax.fori_loop(0, NCH // 2, pair_body, 0)

    tt = NCH - 1
    offt = pl.multiple_of(base_p + tt * CH, 8)
    pltpu.sync_copy(idx_hbm.at[wid, pl.ds(tt - 1, 2)], idx_v)
    cgt1, cgt2, cwt = _chunk_start(h_hbm, wij_hbm, idx_v, 1, offt,
                                   xj_a, wp_a, g_a, w_a)
    cgt1.wait()
    cgt2.wait()
    cwt.wait()
    _mul_rows(xj_a, wp_a)
    cst1, cst2 = _chunk_scatter(agg_sh, idx_v, 1, xj_a, s_a)
    cst1.wait()
    cst2.wait()

    plsc.subcore_barrier()
    pltpu.sync_copy(agg_sh.at[rows], out_hbm.at[cid, rows])


def _sc_aggregate(h, wij_packed, idx_i, idx_j, zeros):
    def arrange(v):
        a = v.reshape(E // WBLK, 2, WBLK // 2)
        lo = a[:, 0, :].reshape(NW, NCH, CH)
        hi = a[:, 1, :].reshape(NW, NCH, CH)
        return lo, hi

    ilo, ihi = arrange(idx_i)
    jlo, jhi = arrange(idx_j)
    idx_pack = jnp.stack([ilo, jlo, ihi, jhi], axis=2)
    mesh = plsc.VectorSubcoreMesh(core_axis_name="c", subcore_axis_name="s")
    k = functools.partial(
        pl.kernel,
        mesh=mesh,
        out_type=jax.ShapeDtypeStruct((NC, NPAD, F), jnp.float32),
        scratch_types=[
            pltpu.VMEM((2, 4, CH), jnp.int32),
            pltpu.VMEM((2 * CH, F), jnp.float32),
            pltpu.VMEM((CH, F), jnp.int32),
            pltpu.VMEM((2 * CH, F), jnp.float32),
            pltpu.VMEM((CH, F), jnp.int32),
            pltpu.VMEM_SHARED((NPAD, F), jnp.float32),
            pltpu.SemaphoreType.DMA,
            pltpu.SemaphoreType.DMA,
            pltpu.SemaphoreType.DMA,
            pltpu.SemaphoreType.DMA,
            pltpu.SemaphoreType.DMA,
            pltpu.SemaphoreType.DMA,
        ],
    )(_sc_body)
    return k(h, wij_packed, idx_pack, zeros)


def _out_body(p0_ref, p1_ref, w1_ref, b1_ref, w2_ref, b2_ref, o_ref):
    a = p0_ref[...] + p1_ref[...]
    t = _smu(jnp.dot(a, w1_ref[...], preferred_element_type=jnp.float32)
             + b1_ref[...])
    o_ref[...] = jnp.dot(t, w2_ref[...],
                         preferred_element_type=jnp.float32) + b2_ref[...]


def _compute_out(p0, p1, w1, b1, w2, b2):
    blk = 2000
    return pl.pallas_call(
        _out_body,
        grid=(N // blk,),
        in_specs=[
            pl.BlockSpec((blk, F), lambda i: (i, 0)),
            pl.BlockSpec((blk, F), lambda i: (i, 0)),
            pl.BlockSpec((F, F), lambda i: (0, 0)),
            pl.BlockSpec((1, F), lambda i: (0, 0)),
            pl.BlockSpec((F, F), lambda i: (0, 0)),
            pl.BlockSpec((1, F), lambda i: (0, 0)),
        ],
        out_specs=pl.BlockSpec((blk, F), lambda i: (i, 0)),
        out_shape=jax.ShapeDtypeStruct((N, F), jnp.float32),
    )(p0, p1, w1, b1, w2, b2)


def kernel(x, f_ij, rcut_ij, W_in2f, W_f1, b_f1, W_f2, b_f2,
           W_o1, b_o1, W_o2, b_o2, idx_i, idx_j):
    h = _compute_h(x, W_in2f)
    wij = _compute_wij(f_ij, rcut_ij.reshape(E, 1),
                       W_f1, b_f1.reshape(1, F), W_f2, b_f2.reshape(1, F))
    zeros = jnp.zeros((NPAD, F), jnp.float32)
    parts = _sc_aggregate(h, wij, idx_i.astype(jnp.int32),
                          idx_j.astype(jnp.int32), zeros)
    out = _compute_out(parts[0], parts[1],
                       W_o1, b_o1.reshape(1, F), W_o2, b_o2.reshape(1, F))
    return out
